# baseline (speedup 1.0000x reference)
import numpy as np
import jax
import jax.numpy as jnp
from jax.experimental import pallas as pl
from jax.experimental.pallas import tpu as pltpu

H = 3
EMB = 256


def _gat(x, src, dst, valid, W, a_s, a_d, b, n):
    sl = jnp.arange(n, dtype=src.dtype)
    src_f = jnp.concatenate([src, sl])
    dst_f = jnp.concatenate([dst, sl])
    val_f = jnp.concatenate([valid, jnp.ones((n,), bool)])
    h = (x @ W).reshape(n, H, EMB)
    e = jax.nn.leaky_relu((h * a_s[None]).sum(-1)[src_f] + (h * a_d[None]).sum(-1)[dst_f], 0.2)
    e = jnp.where(val_f[:, None], e, -1e9)
    m = jax.lax.stop_gradient(jax.ops.segment_max(e, dst_f, num_segments=n))
    ex = jnp.exp(e - m[dst_f]) * val_f[:, None].astype(e.dtype)
    den = jax.ops.segment_sum(ex, dst_f, num_segments=n)
    alpha = ex / jnp.maximum(den[dst_f], 1e-16)
    out = jax.ops.segment_sum(h[src_f] * alpha[:, :, None], dst_f, num_segments=n)
    return out.reshape(n, H * EMB) + b


def _pool(x, src, dst, valid, p, ratio, n):
    K = int(np.ceil(ratio * n))
    score = jnp.tanh((x @ p) / (jnp.linalg.norm(p) + 1e-16))
    vals, perm = jax.lax.top_k(score, K)
    x_new = x[perm] * vals[:, None]
    mapping = jnp.full((n,), K, dtype=src.dtype).at[perm].set(jnp.arange(K, dtype=src.dtype))
    ns = mapping[src]
    nd = mapping[dst]
    nv = valid & (ns < K) & (nd < K)
    return x_new, jnp.minimum(ns, K - 1), jnp.minimum(nd, K - 1), nv, K


def _readout(x):
    return jnp.concatenate([jnp.max(x, axis=0), jnp.mean(x, axis=0)])[None, :]


def _mlp_kernel(z_ref, l1w_ref, l1b_ref, l2w_ref, l2b_ref, o_ref):
    z = z_ref[...]
    h = jnp.maximum(jnp.dot(z, l1w_ref[...], preferred_element_type=jnp.float32)
                    + l1b_ref[...][None, :], 0.0)
    o_ref[...] = jnp.dot(h, l2w_ref[...], preferred_element_type=jnp.float32) \
        + l2b_ref[...][None, :]


def _mlp(z, L1W, L1b, L2W, L2b):
    return pl.pallas_call(
        _mlp_kernel,
        out_shape=jax.ShapeDtypeStruct((z.shape[0], L2W.shape[1]), jnp.float32),
    )(z, L1W, L1b, L2W, L2b)


def kernel(x, _edge_attr, edge_index, batch_index, W1, as1, ad1, b1, Wt1, bt1, p1, W2, as2, ad2, b2, Wt2, bt2, p2, W3, as3, ad3, b3, Wt3, bt3, p3, L1W, L1b, L2W, L2b):
    src = edge_index[0]
    dst = edge_index[1]
    valid = jnp.ones((src.shape[0],), dtype=bool)
    n = x.shape[0]
    h = _gat(x, src, dst, valid, W1, as1, ad1, b1, n)
    h = jax.nn.relu(h @ Wt1 + bt1)
    h, src, dst, valid, n = _pool(h, src, dst, valid, p1, 0.8, n)
    r1 = _readout(h)
    h = _gat(h, src, dst, valid, W2, as2, ad2, b2, n)
    h = jax.nn.relu(h @ Wt2 + bt2)
    h, src, dst, valid, n = _pool(h, src, dst, valid, p2, 0.5, n)
    r2 = _readout(h)
    h = _gat(h, src, dst, valid, W3, as3, ad3, b3, n)
    h = jax.nn.relu(h @ Wt3 + bt3)
    h, src, dst, valid, n = _pool(h, src, dst, valid, p3, 0.2, n)
    r3 = _readout(h)
    z = r1 + r2 + r3
    return _mlp(z, L1W, L1b, L2W, L2b)


# speedup vs baseline: 1.8798x; 1.8798x over previous
import functools
import numpy as np
import jax
import jax.numpy as jnp
from jax import lax
from jax.experimental import pallas as pl
from jax.experimental.pallas import tpu as pltpu
from jax.experimental.pallas import tpu_sc as plsc

H = 3
EMB = 256
NG = 6
GSZ = 128
NC = 2
NS = 16
NW = NC * NS
RB = 400
NMAX = 10240
HF = NMAX // 2
EPAD = 172032


def _mmA_body(x_ref, w_ref, wsd_ref, hgt_ref, hsd_ref):
    xb = x_ref[...]
    h = jnp.dot(xb, w_ref[...], preferred_element_type=jnp.float32)
    for g in range(NG):
        hgt_ref[g] = h[:, g * GSZ:(g + 1) * GSZ]
    hsd_ref[...] = jnp.dot(xb, wsd_ref[...], preferred_element_type=jnp.float32)


def _mmA(x, W, wsd):
    n, din = x.shape
    grid = (n // RB,)
    return pl.pallas_call(
        _mmA_body,
        grid=grid,
        in_specs=[
            pl.BlockSpec((RB, din), lambda i: (i, 0)),
            pl.BlockSpec((din, H * EMB), lambda i: (0, 0)),
            pl.BlockSpec((din, 8), lambda i: (0, 0)),
        ],
        out_specs=[
            pl.BlockSpec((NG, RB, GSZ), lambda i: (0, i, 0)),
            pl.BlockSpec((RB, 8), lambda i: (i, 0)),
        ],
        out_shape=[
            jax.ShapeDtypeStruct((NG, NMAX, GSZ), jnp.float32),
            jax.ShapeDtypeStruct((NMAX, 8), jnp.float32),
        ],
    )(x, W, wsd)


def _mmB_body(un_ref, ivg_ref, wt_ref, btp_ref, p_ref, ps_ref, h_ref, sc_ref):
    acc = jnp.zeros((RB, EMB), jnp.float32)
    for g in range(NG):
        u = un_ref[g]
        iv = ivg_ref[:, g:g + 1]
        acc = acc + jnp.dot(u * iv, wt_ref[g * GSZ:(g + 1) * GSZ, :],
                            preferred_element_type=jnp.float32)
    hr = jnp.maximum(acc + btp_ref[...], 0.0)
    h_ref[...] = hr
    sc_ref[...] = jnp.tanh(
        jnp.dot(hr, p_ref[...], preferred_element_type=jnp.float32)
        / ps_ref[...])


def _mmB(un6, ivg, Wt, btp, p2, ps, n):
    grid = (n // RB,)
    return pl.pallas_call(
        _mmB_body,
        grid=grid,
        in_specs=[
            pl.BlockSpec((NG, RB, GSZ), lambda i: (0, i, 0)),
            pl.BlockSpec((RB, 8), lambda i: (i, 0)),
            pl.BlockSpec((H * EMB, EMB), lambda i: (0, 0)),
            pl.BlockSpec((1, EMB), lambda i: (0, 0)),
            pl.BlockSpec((EMB, 1), lambda i: (0, 0)),
            pl.BlockSpec((1, 1), lambda i: (0, 0)),
        ],
        out_specs=[
            pl.BlockSpec((RB, EMB), lambda i: (i, 0)),
            pl.BlockSpec((RB, 1), lambda i: (i, 0)),
        ],
        out_shape=[
            jax.ShapeDtypeStruct((n, EMB), jnp.float32),
            jax.ShapeDtypeStruct((n, 1), jnp.float32),
        ],
    )(un6, ivg, Wt, btp, p2, ps)


def _readout_body(k, x_ref, o_ref):
    xb = x_ref[...]
    o_ref[0:1, :] = jnp.max(xb, axis=0, keepdims=True)
    o_ref[1:2, :] = jnp.sum(xb, axis=0, keepdims=True) * (1.0 / k)


def _readout(x):
    k = x.shape[0]
    return pl.pallas_call(
        functools.partial(_readout_body, k),
        out_shape=jax.ShapeDtypeStruct((2, EMB), jnp.float32),
    )(x)


def _mlp_kernel(z_ref, l1w_ref, l1b_ref, l2w_ref, l2b_ref, o_ref):
    z = z_ref[...]
    hh = jnp.maximum(jnp.dot(z, l1w_ref[...], preferred_element_type=jnp.float32)
                     + l1b_ref[...][None, :], 0.0)
    o_ref[...] = jnp.dot(hh, l2w_ref[...], preferred_element_type=jnp.float32) \
        + l2b_ref[...][None, :]


def _mlp(z, L1W, L1b, L2W, L2b):
    return pl.pallas_call(
        _mlp_kernel,
        out_shape=jax.ShapeDtypeStruct((z.shape[0], L2W.shape[1]), jnp.float32),
    )(z, L1W, L1b, L2W, L2b)


def _mesh():
    return plsc.VectorSubcoreMesh(core_axis_name="c", subcore_axis_name="s",
                                  num_cores=NC, num_subcores=NS)


@functools.cache
def _sc_passA():
    cha = EPAD // NW
    ta = cha // 16
    dnr = 256

    @functools.partial(
        pl.kernel,
        out_type=(jax.ShapeDtypeStruct((H * EPAD,), jnp.float32),
                  jax.ShapeDtypeStruct((NC, dnr, 128), jnp.float32)),
        mesh=_mesh(),
        compiler_params=pltpu.CompilerParams(needs_layout_passes=False),
        scratch_types=[
            pltpu.VMEM((H * NMAX,), jnp.float32),
            pltpu.VMEM((H * NMAX,), jnp.float32),
            pltpu.VMEM((dnr, 128), jnp.float32),
            pltpu.VMEM((2, 128), jnp.int32),
            pltpu.VMEM((cha,), jnp.int32),
            pltpu.VMEM((cha,), jnp.int32),
            pltpu.VMEM((cha,), jnp.float32),
            pltpu.VMEM((H * cha,), jnp.float32),
            pltpu.VMEM_SHARED((dnr, 128), jnp.float32),
        ],
    )
    def kern(hs_hbm, hd_hbm, src_hbm, dst_hbm, vm_hbm, ex_hbm, den_hbm,
             hs_v, hd_v, den_v, rid_v, src_v, dst_v, vm_v, ex_v, shden):
        c = lax.axis_index("c")
        s = lax.axis_index("s")
        wid = s * NC + c
        off = wid * cha
        pltpu.sync_copy(hs_hbm, hs_v)
        pltpu.sync_copy(hd_hbm, hd_v)
        pltpu.sync_copy(src_hbm.at[pl.ds(off, cha)], src_v)
        pltpu.sync_copy(dst_hbm.at[pl.ds(off, cha)], dst_v)
        pltpu.sync_copy(vm_hbm.at[pl.ds(off, cha)], vm_v)

        def zbody(i, _):
            for f in range(8):
                den_v[i, pl.ds(f * 16, 16)] = jnp.zeros((16,), jnp.float32)
            return 0
        lax.fori_loop(0, dnr, zbody, 0)
        for hb in range(2):
            for f in range(8):
                rid_v[hb, pl.ds(f * 16, 16)] = \
                    lax.iota(jnp.int32, 16) + (hb * 128 + f * 16)

        @pl.when(s == 0)
        def _():
            pltpu.sync_copy(den_v, shden)
        plsc.subcore_barrier()

        def ebody(i, _):
            s16 = src_v[pl.ds(i * 16, 16)]
            d16 = dst_v[pl.ds(i * 16, 16)]
            vm16 = vm_v[pl.ds(i * 16, 16)]
            for k in range(H):
                av = plsc.load_gather(hs_v, [s16 + k * NMAX])
                bv = plsc.load_gather(hd_v, [d16 + k * NMAX])
                e = av + bv
                e = jnp.maximum(e, 0.2 * e)
                exv = jnp.exp(e) * vm16
                ex_v[pl.ds(k * cha + i * 16, 16)] = exv
                di = d16 + k * NMAX
                plsc.addupdate_scatter(
                    den_v, [lax.shift_right_logical(di, 7), di & 127], exv)
            return 0
        lax.fori_loop(0, ta, ebody, 0)

        for k in range(H):
            pltpu.sync_copy(ex_v.at[pl.ds(k * cha, cha)],
                            ex_hbm.at[pl.ds(k * EPAD + off, cha)])
        pltpu.sync_copy(den_v.at[pl.ds(0, 128)],
                        shden.at[rid_v.at[0]], add=True)
        pltpu.sync_copy(den_v.at[pl.ds(128, 128)],
                        shden.at[rid_v.at[1]], add=True)
        plsc.subcore_barrier()

        @pl.when(s == 0)
        def _():
            pltpu.sync_copy(shden, den_hbm.at[c])

    return kern


@functools.cache
def _sc_passB():
    chb = EPAD // NS
    nb = chb // 128
    rpt = HF // NS
    full_blocks = rpt // 64
    rem = rpt % 64

    @functools.partial(
        pl.kernel,
        out_type=jax.ShapeDtypeStruct((NG * NMAX, GSZ), jnp.float32),
        mesh=_mesh(),
        compiler_params=pltpu.CompilerParams(needs_layout_passes=False),
        scratch_types=[
            pltpu.VMEM((chb,), jnp.int32),
            pltpu.VMEM((chb,), jnp.int32),
            pltpu.VMEM((chb,), jnp.int32),
            pltpu.VMEM((nb, 128), jnp.int32),
            pltpu.VMEM((chb,), jnp.float32),
            pltpu.VMEM((128, GSZ), jnp.float32),
            pltpu.VMEM((64, GSZ), jnp.float32),
            pltpu.VMEM_SHARED((HF, GSZ), jnp.float32),
        ],
    )
    def kern(hgt_hbm, ex_hbm, src_hbm, dst_hbm, un_hbm,
             src_raw, dst_raw, src_i, dst_i, ex_v, rows_v, zbuf, acc):
        c = lax.axis_index("c")
        s = lax.axis_index("s")
        off = s * chb
        row0 = s * rpt
        pltpu.sync_copy(src_hbm.at[pl.ds(off, chb)], src_raw)
        pltpu.sync_copy(dst_hbm.at[pl.ds(off, chb)], dst_raw)

        def zb(i, _):
            for f in range(8):
                zbuf[i, pl.ds(f * 16, 16)] = jnp.zeros((16,), jnp.float32)
            return 0
        lax.fori_loop(0, 64, zb, 0)

        for gi in range(NG // NC):
            g = 3 * c + gi
            k = g // 2

            def sbody(i, _):
                s16 = src_raw[pl.ds(i * 16, 16)]
                src_i[pl.ds(i * 16, 16)] = s16 + g * NMAX
                return 0
            lax.fori_loop(0, chb // 16, sbody, 0)
            pltpu.sync_copy(ex_hbm.at[pl.ds(k * EPAD + off, chb)], ex_v)

            for hf in range(2):
                lo = hf * HF

                def dbody(jr, _):
                    for f in range(8):
                        d16 = dst_raw[pl.ds(jr * 128 + f * 16, 16)]
                        dst_i[jr, pl.ds(f * 16, 16)] = \
                            jnp.clip(d16 - lo, 0, HF - 1)
                    return 0
                lax.fori_loop(0, nb, dbody, 0)

                plsc.subcore_barrier()
                for t in range(full_blocks):
                    pltpu.sync_copy(zbuf, acc.at[pl.ds(row0 + t * 64, 64)])
                if rem:
                    pltpu.sync_copy(zbuf.at[pl.ds(0, rem)],
                                    acc.at[pl.ds(row0 + full_blocks * 64,
                                                 rem)])
                plsc.subcore_barrier()

                def bbody(j, _):
                    pltpu.sync_copy(hgt_hbm.at[src_i.at[pl.ds(j * 128, 128)]],
                                    rows_v)

                    def qbody(q, _):
                        base = j * 128 + q * 16
                        exv = ex_v[pl.ds(base, 16)]
                        d16 = dst_raw[pl.ds(base, 16)]
                        inh = (d16 >= lo) & (d16 < lo + HF)
                        exv = jnp.where(inh, exv, 0.0)
                        for i in range(16):
                            b = exv[i]
                            r = q * 16 + i
                            for f in range(8):
                                rows_v[r, pl.ds(f * 16, 16)] = \
                                    rows_v[r, pl.ds(f * 16, 16)] * b
                        return 0
                    lax.fori_loop(0, 8, qbody, 0)
                    pltpu.sync_copy(rows_v, acc.at[dst_i.at[j]], add=True)
                    return 0
                lax.fori_loop(0, nb, bbody, 0)
                plsc.subcore_barrier()
                pltpu.sync_copy(acc.at[pl.ds(row0, rpt)],
                                un_hbm.at[pl.ds(g * NMAX + lo + row0, rpt)])

    return kern


def _gat_layer(x_cur, s_base, d_base, W, a_s, a_d, b, Wt, bt, p, n):
    din = x_cur.shape[1]
    w3 = W.reshape(din, H, EMB)
    Ws = jnp.einsum('dkc,kc->dk', w3, a_s)
    Wd = jnp.einsum('dkc,kc->dk', w3, a_d)
    wsd = jnp.concatenate([Ws, Wd, jnp.zeros((din, 2), jnp.float32)], axis=1)
    hgt6, hsd = _mmA(x_cur, W, wsd)
    hgt = hgt6.reshape(NG * NMAX, GSZ)
    hs_f = hsd[:, 0:H].T.reshape(H * NMAX)
    hd_f = hsd[:, H:2 * H].T.reshape(H * NMAX)

    e_tot = s_base.shape[0] + n
    sl = jnp.arange(n, dtype=jnp.int32)
    pad = jnp.full((EPAD - e_tot,), n, jnp.int32)
    src_f = jnp.concatenate([s_base, sl, pad])
    dst_f = jnp.concatenate([d_base, sl, pad])
    vmask = ((src_f < n) & (dst_f < n)).astype(jnp.float32)
    src_c = jnp.minimum(src_f, n - 1)
    dst_c = jnp.minimum(dst_f, n - 1)

    ex, den_parts = _sc_passA()(hs_f, hd_f, src_c, dst_c, vmask)
    unnorm = _sc_passB()(hgt, ex, src_c, dst_c)

    den = den_parts.sum(axis=0).reshape(-1)[:H * NMAX].reshape(H, NMAX)
    inv = 1.0 / jnp.maximum(den, 1e-16)
    ivg = inv[jnp.array([0, 0, 1, 1, 2, 2, 0, 0]), :].T
    btp = (bt + b @ Wt).reshape(1, EMB)
    ps = (jnp.linalg.norm(p) + 1e-16).reshape(1, 1)
    hrelu, score = _mmB(unnorm.reshape(NG, NMAX, GSZ), ivg, Wt, btp,
                        p.reshape(EMB, 1), ps, n)
    return hrelu, score[:, 0]


def kernel(x, _edge_attr, edge_index, batch_index,
           W1, as1, ad1, b1, Wt1, bt1, p1,
           W2, as2, ad2, b2, Wt2, bt2, p2,
           W3, as3, ad3, b3, Wt3, bt3, p3,
           L1W, L1b, L2W, L2b):
    s_base = edge_index[0]
    d_base = edge_index[1]
    n = x.shape[0]
    params = [(W1, as1, ad1, b1, Wt1, bt1, p1, 0.8),
              (W2, as2, ad2, b2, Wt2, bt2, p2, 0.5),
              (W3, as3, ad3, b3, Wt3, bt3, p3, 0.2)]
    x_cur = x
    rs = []
    for (W, a_s, a_d, b, Wt, bt, p, ratio) in params:
        hrelu, score = _gat_layer(x_cur, s_base, d_base,
                                  W, a_s, a_d, b, Wt, bt, p, n)
        k = int(np.ceil(ratio * n))
        vals, perm = lax.top_k(score, k)
        x_cur = hrelu[perm] * vals[:, None]
        rs.append(_readout(x_cur))
        mapping = jnp.full((n,), k, jnp.int32).at[perm].set(
            jnp.arange(k, dtype=jnp.int32))
        s_base = jnp.where(s_base < n,
                           jnp.take(mapping, jnp.minimum(s_base, n - 1)), k)
        d_base = jnp.where(d_base < n,
                           jnp.take(mapping, jnp.minimum(d_base, n - 1)), k)
        n = k
    z = (rs[0] + rs[1] + rs[2]).reshape(1, 2 * EMB)
    return _mlp(z, L1W, L1b, L2W, L2b)
